# baseline (speedup 1.0000x reference)
import functools

import jax
import jax.numpy as jnp
from jax import lax
from jax.experimental import pallas as pl
from jax.experimental.pallas import tpu as pltpu
from jax.experimental.pallas import tpu_sc as plsc

B, L = 4096, 50
D = 64
LPAD, DPAD = 56, 128
N = B * L
NC, NS = 2, 16
NW = NC * NS
RPW = N // NW
K = 4
CC = K * L
NCHUNK = RPW // CC
P = 8

_mesh = plsc.VectorSubcoreMesh(core_axis_name="c", subcore_axis_name="s")


@functools.partial(
    pl.kernel,
    mesh=_mesh,
    out_type=jax.ShapeDtypeStruct((B, LPAD, DPAD), jnp.float32),
    compiler_params=pltpu.CompilerParams(use_tc_tiling_on_sc=False),
    scratch_types=[
        pltpu.VMEM((RPW,), jnp.int32),
        pltpu.VMEM((RPW,), jnp.int32),
        pltpu.VMEM((RPW,), jnp.int32),
        pltpu.VMEM((P, CC, D), jnp.float32),
    ] + [pltpu.SemaphoreType.DMA] * 24,
)
def _triple_embed(oid, tid, cid, t1, t2, t3, out,
                  i1, i2, i3, acc, *sems):
    wid = lax.axis_index("s") * NC + lax.axis_index("c")
    wb = wid * RPW

    pltpu.sync_copy(oid.at[pl.ds(wb, RPW)], i1)
    pltpu.sync_copy(tid.at[pl.ds(wb, RPW)], i2)
    pltpu.sync_copy(cid.at[pl.ds(wb, RPW)], i3)

    sa = sems[0:P]
    sb = sems[P:2 * P]
    sd = sems[2 * P:]

    def f1(c, p):
        pltpu.async_copy(t1.at[i1.at[pl.ds(c * CC, CC)]], acc.at[p], sa[p])

    def f23(c, p):
        isl = pl.ds(c * CC, CC)
        pltpu.make_async_copy(t1.at[i1.at[isl]], acc.at[p], sa[p]).wait()
        pltpu.async_copy(t2.at[i2.at[isl]], acc.at[p], sb[p], add=True)
        pltpu.async_copy(t3.at[i3.at[isl]], acc.at[p], sb[p], add=True)

    def wbf(c, p):
        isl = pl.ds(c * CC, CC)
        pltpu.make_async_copy(t2.at[i2.at[isl]], acc.at[p], sb[p]).wait()
        pltpu.make_async_copy(t3.at[i3.at[isl]], acc.at[p], sb[p]).wait()
        bb = wid * (B // NW) + c * K
        for j in range(K):
            pltpu.async_copy(acc.at[p, pl.ds(j * L, L)],
                             out.at[bb + j, pl.ds(0, L), pl.ds(0, D)], sd[p])

    def wbw(c, p):
        bb = wid * (B // NW) + c * K
        for j in range(K):
            pltpu.make_async_copy(acc.at[p, pl.ds(j * L, L)],
                                  out.at[bb + j, pl.ds(0, L), pl.ds(0, D)],
                                  sd[p]).wait()

    for s in range(P):
        f1(s, s)
        if s >= 3:
            f23(s - 3, s - 3)
        if s >= 5:
            wbf(s - 5, s - 5)
        if s >= 7:
            wbw(s - 7, s - 7)

    def body(h, carry):
        s0 = P * h
        for q in range(P):
            s = s0 + q
            wbw(s - 7, (q + 1) % P)
            f1(s, q)
            f23(s - 3, (q + 5) % P)
            wbf(s - 5, (q + 3) % P)
        return carry

    lax.fori_loop(1, NCHUNK // P, body, 0)

    for s in range(NCHUNK, NCHUNK + 7):
        if s - 7 < NCHUNK:
            wbw(s - 7, (s - 7) % P)
        if s - 3 < NCHUNK:
            f23(s - 3, (s - 3) % P)
        if s - 5 < NCHUNK:
            wbf(s - 5, (s - 5) % P)


def kernel(out_ids, tree_ids, ctx_ids, out_table, tree_table, ctx_table):
    oid = out_ids.reshape(-1).astype(jnp.int32)
    tid = tree_ids.reshape(-1).astype(jnp.int32)
    cid = ctx_ids.reshape(-1).astype(jnp.int32)
    res = _triple_embed(oid, tid, cid, out_table, tree_table, ctx_table)
    return lax.slice(res, (0, 0, 0), (B, L, D))
